# baseline (speedup 1.0000x reference)
import jax
import jax.numpy as jnp
from jax.experimental import pallas as pl
from jax.experimental.pallas import tpu as pltpu
from jax.experimental.pallas import tpu_sc as plsc

K = 8192
D = 256
N = 8 * 32 * 32
NB = 256
NBLK = N // NB
BLK_A = 24
ROWS_A = BLK_A * NB
ROWS_B = N - ROWS_A
GW = 128
COMMIT = 0.25


def _vq_body(x_ref, x2_ref, et2_ref, e2_ref, iota_ref, ones_ref,
             idx_ref, enc_ref, counts_ref):
    i = pl.program_id(0)
    mm2 = jnp.dot(x_ref[...], et2_ref[...], preferred_element_type=jnp.float32)
    d = (x2_ref[...] + e2_ref[...]) - mm2
    vmin = jnp.min(d, axis=1, keepdims=True)
    iotaf = iota_ref[...]
    idxf = jnp.min(jnp.where(d == vmin, iotaf, float(K)), axis=1)
    idx_ref[...] = idxf.astype(jnp.int32).reshape(1, NB)
    enc = jnp.where(iotaf == idxf[:, None], 1.0, 0.0)
    enc_ref[...] = enc

    @pl.when(i == 0)
    def _():
        counts_ref[...] = jnp.zeros_like(counts_ref)

    counts_ref[...] += jnp.dot(ones_ref[...], enc,
                               preferred_element_type=jnp.float32)


def _vq_body_alias(x_ref, x2_ref, et2_ref, e2_ref, iota_ref, ones_ref,
                   _enc_prev, idx_ref, enc_ref, counts_ref):
    _vq_body(x_ref, x2_ref, et2_ref, e2_ref, iota_ref, ones_ref,
             idx_ref, enc_ref, counts_ref)


def _main_call(flat, x2, et2, e2, iotaf, ones_row, off_blk, nblk,
               enc_prev=None):
    rows = nblk * NB
    in_specs = [
        pl.BlockSpec((NB, D), lambda i: (i + off_blk, 0)),
        pl.BlockSpec((NB, 1), lambda i: (i + off_blk, 0)),
        pl.BlockSpec((D, K), lambda i: (0, 0)),
        pl.BlockSpec((1, K), lambda i: (0, 0)),
        pl.BlockSpec((1, K), lambda i: (0, 0)),
        pl.BlockSpec((1, NB), lambda i: (0, 0)),
    ]
    args = [flat, x2, et2, e2, iotaf, ones_row]
    body = _vq_body
    aliases = {}
    if enc_prev is not None:
        in_specs.append(pl.BlockSpec(memory_space=pl.ANY))
        args.append(enc_prev)
        body = _vq_body_alias
        aliases = {6: 1}
    return pl.pallas_call(
        body,
        grid=(nblk,),
        in_specs=in_specs,
        out_specs=[
            pl.BlockSpec((1, NB), lambda i: (0, i)),
            pl.BlockSpec((NB, K), lambda i: (i + off_blk, 0)),
            pl.BlockSpec((1, K), lambda i: (0, 0)),
        ],
        out_shape=[
            jax.ShapeDtypeStruct((1, rows), jnp.int32),
            jax.ShapeDtypeStruct((N, K), jnp.float32),
            jax.ShapeDtypeStruct((1, K), jnp.float32),
        ],
        input_output_aliases=aliases,
        compiler_params=pltpu.CompilerParams(
            dimension_semantics=("arbitrary",)),
    )(*args)


def _make_sc_gather(rows):
    def _sc_gather(emb_hbm, i_hbm, o_hbm):
        def body(i_vmem, o_vmem):
            pltpu.sync_copy(emb_hbm.at[i_vmem.at[0]], o_vmem)

        pltpu.emit_pipeline(
            body,
            grid=(rows // GW,),
            in_specs=[pl.BlockSpec((1, GW), index_map=lambda i: (0, i))],
            out_specs=[pl.BlockSpec((GW, D), index_map=lambda i: (i, 0))],
            core_axis_name=("core", "subcore"),
            dimension_semantics=(pltpu.PARALLEL,),
        )(i_hbm, o_hbm)

    return _sc_gather


def _finalize(x_ref, qa_ref, qb_ref, ca_ref, cb_ref,
              qst_ref, loss_ref, perp_ref):
    xa = x_ref[:ROWS_A, :]
    xb = x_ref[ROWS_A:, :]
    dqa = qa_ref[...] - xa
    dqb = qb_ref[...] - xb
    qst_ref[:ROWS_A, :] = xa + dqa
    qst_ref[ROWS_A:, :] = xb + dqb
    sse = jnp.sum(dqa * dqa) + jnp.sum(dqb * dqb)
    mse = sse * (1.0 / (N * D))
    loss_ref[...] = (mse + COMMIT * mse).reshape(1, 1)
    p = (ca_ref[...] + cb_ref[...]) * (1.0 / N)
    ent = jnp.sum(p * jnp.log(p + 1e-10))
    perp_ref[...] = jnp.exp(-ent).reshape(1, 1)


def kernel(inputs, embedding, reset):
    del reset
    x = jnp.transpose(inputs, (0, 2, 3, 1))
    input_shape = x.shape
    flat = x.reshape(-1, D)
    x2 = jnp.sum(flat ** 2, axis=1, keepdims=True)
    e2 = jnp.sum(embedding ** 2, axis=1).reshape(1, K)
    et2 = embedding.T * 2.0
    iotaf = jnp.arange(K, dtype=jnp.float32).reshape(1, K)
    ones_row = jnp.ones((1, NB), jnp.float32)

    idx_a, enc_a, counts_a = _main_call(
        flat, x2, et2, e2, iotaf, ones_row, 0, BLK_A)
    idx_b, enc, counts_b = _main_call(
        flat, x2, et2, e2, iotaf, ones_row, BLK_A, NBLK - BLK_A,
        enc_prev=enc_a)

    sc_mesh = plsc.VectorSubcoreMesh(
        core_axis_name="core", subcore_axis_name="subcore")
    q_a = pl.kernel(
        _make_sc_gather(ROWS_A),
        out_type=jax.ShapeDtypeStruct((ROWS_A, D), jnp.float32),
        mesh=sc_mesh,
    )(embedding, idx_a)
    q_b = pl.kernel(
        _make_sc_gather(ROWS_B),
        out_type=jax.ShapeDtypeStruct((ROWS_B, D), jnp.float32),
        mesh=sc_mesh,
    )(embedding, idx_b)

    qst, loss, perp = pl.pallas_call(
        _finalize,
        in_specs=[
            pl.BlockSpec((N, D), lambda: (0, 0)),
            pl.BlockSpec((ROWS_A, D), lambda: (0, 0)),
            pl.BlockSpec((ROWS_B, D), lambda: (0, 0)),
            pl.BlockSpec((1, K), lambda: (0, 0)),
            pl.BlockSpec((1, K), lambda: (0, 0)),
        ],
        out_specs=[
            pl.BlockSpec((N, D), lambda: (0, 0)),
            pl.BlockSpec((1, 1), lambda: (0, 0)),
            pl.BlockSpec((1, 1), lambda: (0, 0)),
        ],
        out_shape=[
            jax.ShapeDtypeStruct((N, D), jnp.float32),
            jax.ShapeDtypeStruct((1, 1), jnp.float32),
            jax.ShapeDtypeStruct((1, 1), jnp.float32),
        ],
    )(flat, q_a, q_b, counts_a, counts_b)

    loss = loss[0, 0]
    perplexity = perp[0, 0]
    qst_nchw = jnp.transpose(qst.reshape(input_shape), (0, 3, 1, 2))
    return (loss, qst_nchw, perplexity, enc)


# speedup vs baseline: 1.0584x; 1.0584x over previous
import jax
import jax.numpy as jnp
from jax.experimental import pallas as pl
from jax.experimental.pallas import tpu as pltpu
from jax.experimental.pallas import tpu_sc as plsc

K = 8192
D = 256
N = 8 * 32 * 32
NB = 256
NBLK = N // NB
BLK_A = 24
ROWS_A = BLK_A * NB
ROWS_B = N - ROWS_A
GW = 128
COMMIT = 0.25


def _vq_body(x_ref, x2_ref, et2_ref, e2_ref, iota_ref, ones_ref,
             idx_ref, enc_ref, counts_ref):
    i = pl.program_id(0)
    mm2 = jnp.dot(x_ref[...], et2_ref[...], preferred_element_type=jnp.float32)
    d = (x2_ref[...] + e2_ref[...]) - mm2
    vmin = jnp.min(d, axis=1, keepdims=True)
    iotaf = iota_ref[...]
    idxf = jnp.min(jnp.where(d == vmin, iotaf, float(K)), axis=1)
    idx_ref[...] = idxf.astype(jnp.int32).reshape(1, NB)
    enc = jnp.where(iotaf == idxf[:, None], 1.0, 0.0)
    enc_ref[...] = enc

    @pl.when(i == 0)
    def _():
        counts_ref[...] = jnp.zeros_like(counts_ref)

    counts_ref[...] += jnp.dot(ones_ref[...], enc,
                               preferred_element_type=jnp.float32)


def _vq_body_alias(x_ref, x2_ref, et2_ref, e2_ref, iota_ref, ones_ref,
                   _enc_prev, idx_ref, enc_ref, counts_ref):
    _vq_body(x_ref, x2_ref, et2_ref, e2_ref, iota_ref, ones_ref,
             idx_ref, enc_ref, counts_ref)


def _main_call(flat, x2, et2, e2, iotaf, ones_row, off_blk, nblk,
               enc_prev=None):
    rows = nblk * NB
    in_specs = [
        pl.BlockSpec((NB, D), lambda i: (i + off_blk, 0)),
        pl.BlockSpec((NB, 1), lambda i: (i + off_blk, 0)),
        pl.BlockSpec((D, K), lambda i: (0, 0)),
        pl.BlockSpec((1, K), lambda i: (0, 0)),
        pl.BlockSpec((1, K), lambda i: (0, 0)),
        pl.BlockSpec((1, NB), lambda i: (0, 0)),
    ]
    args = [flat, x2, et2, e2, iotaf, ones_row]
    body = _vq_body
    aliases = {}
    if enc_prev is not None:
        in_specs.append(pl.BlockSpec(memory_space=pl.ANY))
        args.append(enc_prev)
        body = _vq_body_alias
        aliases = {6: 1}
    return pl.pallas_call(
        body,
        grid=(nblk,),
        in_specs=in_specs,
        out_specs=[
            pl.BlockSpec((1, NB), lambda i: (0, i)),
            pl.BlockSpec((NB, K), lambda i: (i + off_blk, 0)),
            pl.BlockSpec((1, K), lambda i: (0, 0)),
        ],
        out_shape=[
            jax.ShapeDtypeStruct((1, rows), jnp.int32),
            jax.ShapeDtypeStruct((N, K), jnp.float32),
            jax.ShapeDtypeStruct((1, K), jnp.float32),
        ],
        input_output_aliases=aliases,
        compiler_params=pltpu.CompilerParams(
            dimension_semantics=("arbitrary",)),
    )(*args)


def _make_sc_gather(rows):
    def _sc_gather(emb_hbm, i_hbm, o_hbm):
        def body(i_vmem, o_vmem):
            pltpu.sync_copy(emb_hbm.at[i_vmem.at[0]], o_vmem)

        pltpu.emit_pipeline(
            body,
            grid=(rows // GW,),
            in_specs=[pl.BlockSpec((1, GW), index_map=lambda i: (0, i))],
            out_specs=[pl.BlockSpec((GW, D), index_map=lambda i: (i, 0))],
            core_axis_name=("core", "subcore"),
            dimension_semantics=(pltpu.PARALLEL,),
        )(i_hbm, o_hbm)

    return _sc_gather


def _finalize(x_ref, q_ref, counts_ref, qst_ref, loss_ref, perp_ref):
    x = x_ref[...]
    q = q_ref[...]
    dq = q - x
    qst_ref[...] = x + dq
    mse = jnp.sum(dq * dq) * (1.0 / (N * D))
    loss_ref[...] = (mse + COMMIT * mse).reshape(1, 1)
    p = counts_ref[...] * (1.0 / N)
    ent = jnp.sum(p * jnp.log(p + 1e-10))
    perp_ref[...] = jnp.exp(-ent).reshape(1, 1)


def kernel(inputs, embedding, reset):
    del reset
    x = jnp.transpose(inputs, (0, 2, 3, 1))
    input_shape = x.shape
    flat = x.reshape(-1, D)
    x2 = jnp.sum(flat ** 2, axis=1, keepdims=True)
    e2 = jnp.sum(embedding ** 2, axis=1).reshape(1, K)
    et2 = embedding.T * 2.0
    iotaf = jnp.arange(K, dtype=jnp.float32).reshape(1, K)
    ones_row = jnp.ones((1, NB), jnp.float32)

    idx, enc, counts = _main_call(
        flat, x2, et2, e2, iotaf, ones_row, 0, NBLK)

    sc_mesh = plsc.VectorSubcoreMesh(
        core_axis_name="core", subcore_axis_name="subcore")
    quantized = pl.kernel(
        _make_sc_gather(N),
        out_type=jax.ShapeDtypeStruct((N, D), jnp.float32),
        mesh=sc_mesh,
    )(embedding, idx)

    qst, loss, perp = pl.pallas_call(
        _finalize,
        in_specs=[
            pl.BlockSpec((N, D), lambda: (0, 0)),
            pl.BlockSpec((N, D), lambda: (0, 0)),
            pl.BlockSpec((1, K), lambda: (0, 0)),
        ],
        out_specs=[
            pl.BlockSpec((N, D), lambda: (0, 0)),
            pl.BlockSpec((1, 1), lambda: (0, 0)),
            pl.BlockSpec((1, 1), lambda: (0, 0)),
        ],
        out_shape=[
            jax.ShapeDtypeStruct((N, D), jnp.float32),
            jax.ShapeDtypeStruct((1, 1), jnp.float32),
            jax.ShapeDtypeStruct((1, 1), jnp.float32),
        ],
    )(flat, quantized, counts)

    loss = loss[0, 0]
    perplexity = perp[0, 0]
    qst_nchw = jnp.transpose(qst.reshape(input_shape), (0, 3, 1, 2))
    return (loss, qst_nchw, perplexity, enc)
